# baseline (speedup 1.0000x reference)
import jax
import jax.numpy as jnp
from jax.experimental import pallas as pl
from jax.experimental.pallas import tpu as pltpu

_TM = 1024
_NS = 4


def _router_body(x0, x1, x2, x3, w_ref, b_ref, o_ref):
    kh = x0.shape[1]
    wb = w_ref[...].astype(jnp.bfloat16)
    dn = (((1,), (1,)), ((), ()))
    acc = b_ref[...].astype(jnp.float32)
    parts = []
    for s, xr in enumerate((x0, x1, x2, x3)):
        parts.append(jax.lax.dot_general(
            xr[...].astype(jnp.bfloat16), wb[:, s * kh:(s + 1) * kh],
            dimension_numbers=dn, preferred_element_type=jnp.float32))
    o_ref[...] = ((parts[0] + parts[1]) + (parts[2] + parts[3])) + acc


def kernel(hidden_states, W, b):
    tokens, hidden = hidden_states.shape
    experts = W.shape[0]
    kh = hidden // _NS
    b2 = b.reshape(1, experts)
    xspecs = [
        pl.BlockSpec((_TM, kh), lambda i, s=s: (i, s)) for s in range(_NS)
    ]
    return pl.pallas_call(
        _router_body,
        grid=(tokens // _TM,),
        in_specs=xspecs + [
            pl.BlockSpec((experts, hidden), lambda i: (0, 0)),
            pl.BlockSpec((1, experts), lambda i: (0, 0)),
        ],
        out_specs=pl.BlockSpec((_TM, experts), lambda i: (i, 0)),
        out_shape=jax.ShapeDtypeStruct((tokens, experts), jnp.float32),
    )(*([hidden_states] * _NS), W, b2)
